# baseline (speedup 1.0000x reference)
import functools

import jax
import jax.numpy as jnp
from jax import lax
from jax.experimental import pallas as pl
from jax.experimental.pallas import tpu as pltpu
from jax.experimental.pallas import tpu_sc as plsc

NUM_CLASSES = 1000
FEATURE_DIM = 1024
BATCH = 4096
TOPK = 8
TEMP = 0.07
CPAD = 1024
ROW_BLK = 2048
NWORKERS = 32
RPW = BATCH // NWORKERS


def _sim_body(conf_ref, p_ref, f_ref, hc_ref, e_ref, pn_ref):
    i = pl.program_id(0)

    @pl.when(i == 0)
    def _prep():
        conf = conf_ref[...]
        cols = lax.broadcasted_iota(jnp.int32, conf.shape, 1)
        cur = conf
        idxs = []
        for _ in range(TOPK):
            m = jnp.max(cur, axis=1, keepdims=True)
            idx = jnp.min(jnp.where(cur == m, cols, 2 * CPAD), axis=1,
                          keepdims=True)
            idxs.append(idx)
            cur = jnp.where(cols == idx, float("-inf"), cur)
        hc = jnp.concatenate(idxs, axis=1)
        hct = jnp.transpose(hc)
        hc_ref[...] = jnp.concatenate(
            [hct, jnp.zeros((TOPK, CPAD - NUM_CLASSES), jnp.int32)], axis=1)
        p = p_ref[...]
        n = jnp.sqrt(jnp.sum(p * p, axis=1, keepdims=True))
        pn_ref[0:NUM_CLASSES, :] = p / jnp.maximum(n, 1e-12)
        pn_ref[NUM_CLASSES:CPAD, :] = jnp.zeros(
            (CPAD - NUM_CLASSES, FEATURE_DIM), jnp.float32)

    @pl.when(i > 0)
    def _sim():
        f = f_ref[...]
        n = jnp.sqrt(jnp.sum(f * f, axis=1, keepdims=True))
        fn = f * (1.0 / jnp.maximum(n, 1e-12))
        s = lax.dot_general(fn, pn_ref[...], (((1,), (1,)), ((), ())),
                            preferred_element_type=jnp.float32)
        for g in range(CPAD // 128):
            e_ref[:, g, :] = s[:, g * 128:(g + 1) * 128]


def _sc_gather_body(e_hbm, lab_hbm, hc_hbm, out_hbm, lab_v, hc_v, o_v,
                    idx_refs, val_refs, sem):
    wid = lax.axis_index("s") * 2 + lax.axis_index("c")
    base = wid * RPW
    pltpu.sync_copy(hc_hbm, hc_v)
    pltpu.sync_copy(lab_hbm.at[pl.ds(base, RPW)], lab_v)
    iota = lax.broadcasted_iota(jnp.int32, (16,), 0)
    for g in range(RPW // 16):
        grow = g * 16
        labv = lab_v[pl.ds(grow, 16)]
        rowbase = (base + grow + iota) * CPAD
        idx_refs[0][pl.ds(grow, 16)] = rowbase + labv
        for j in range(TOPK):
            colj = plsc.load_gather(hc_v, [j * CPAD + labv])
            idx_refs[1 + j][pl.ds(grow, 16)] = rowbase + colj
    descs = [pltpu.async_copy(e_hbm.at[idx_refs[j]], val_refs[j], sem)
             for j in range(1 + TOPK)]
    for d in descs:
        d.wait()
    for g in range(RPW // 16):
        grow = g * 16
        pos = jnp.exp(val_refs[0][pl.ds(grow, 16)] * (1.0 / TEMP))
        acc = jnp.exp(val_refs[1][pl.ds(grow, 16)] * (1.0 / TEMP))
        for j in range(2, 1 + TOPK):
            acc = acc + jnp.exp(val_refs[j][pl.ds(grow, 16)] * (1.0 / TEMP))
        o_v[pl.ds(grow, 16)] = pos / (pos + acc * (1.0 / TOPK))
    pltpu.sync_copy(o_v, out_hbm.at[wid])


def _loss_body(g_ref, out_ref):
    loss = -jnp.log(g_ref[...])
    out_ref[...] = (jnp.sum(loss) * (1.0 / BATCH)).reshape(1, 1)


def kernel(features, labels, prototypes, confusion_matrix):
    nblk = BATCH // ROW_BLK
    hct, e3 = pl.pallas_call(
        _sim_body,
        grid=(1 + nblk,),
        in_specs=[
            pl.BlockSpec((NUM_CLASSES, NUM_CLASSES), lambda i: (0, 0)),
            pl.BlockSpec((NUM_CLASSES, FEATURE_DIM), lambda i: (0, 0)),
            pl.BlockSpec((ROW_BLK, FEATURE_DIM),
                         lambda i: (jnp.maximum(i - 1, 0), 0)),
        ],
        out_specs=[
            pl.BlockSpec((TOPK, CPAD), lambda i: (0, 0)),
            pl.BlockSpec((ROW_BLK, CPAD // 128, 128),
                         lambda i: (jnp.maximum(i - 1, 0), 0, 0)),
        ],
        out_shape=[jax.ShapeDtypeStruct((TOPK, CPAD), jnp.int32),
                   jax.ShapeDtypeStruct((BATCH, CPAD // 128, 128),
                                        jnp.float32)],
        scratch_shapes=[pltpu.VMEM((CPAD, FEATURE_DIM), jnp.float32)],
    )(confusion_matrix, prototypes, features)
    sc_gather = functools.partial(
        pl.kernel,
        mesh=plsc.VectorSubcoreMesh(core_axis_name="c", subcore_axis_name="s"),
        compiler_params=pltpu.CompilerParams(needs_layout_passes=False),
        out_type=jax.ShapeDtypeStruct((NWORKERS, RPW), jnp.float32),
        scratch_types=[
            pltpu.VMEM((RPW,), jnp.int32),
            pltpu.VMEM((TOPK * CPAD,), jnp.int32),
            pltpu.VMEM((RPW,), jnp.float32),
            [pltpu.VMEM((RPW,), jnp.int32) for _ in range(1 + TOPK)],
            [pltpu.VMEM((RPW,), jnp.float32) for _ in range(1 + TOPK)],
            pltpu.SemaphoreType.DMA,
        ],
    )(_sc_gather_body)
    ratio = sc_gather(e3.reshape(BATCH * CPAD), labels.astype(jnp.int32),
                      hct.reshape(TOPK * CPAD))
    loss = pl.pallas_call(
        _loss_body,
        out_shape=jax.ShapeDtypeStruct((1, 1), jnp.float32),
    )(ratio)
    return loss[0, 0]


# speedup vs baseline: 1.0561x; 1.0561x over previous
import functools

import jax
import jax.numpy as jnp
from jax import lax
from jax.experimental import pallas as pl
from jax.experimental.pallas import tpu as pltpu
from jax.experimental.pallas import tpu_sc as plsc

NUM_CLASSES = 1000
FEATURE_DIM = 1024
BATCH = 4096
TOPK = 8
TEMP = 0.07
CPAD = 1024
ROW_BLK = 1024
NWORKERS = 32
RPW = BATCH // NWORKERS


def _sim_body(conf_ref, p_ref, f_ref, hc_ref, e_ref, pn_ref):
    i = pl.program_id(0)

    @pl.when(i == 0)
    def _prep():
        conf = conf_ref[...]
        cols = lax.broadcasted_iota(jnp.int32, conf.shape, 1)
        cur = conf
        idxs = []
        for _ in range(TOPK):
            m = jnp.max(cur, axis=1, keepdims=True)
            idx = jnp.min(jnp.where(cur == m, cols, 2 * CPAD), axis=1,
                          keepdims=True)
            idxs.append(idx)
            cur = jnp.where(cols == idx, float("-inf"), cur)
        hc = jnp.concatenate(idxs, axis=1)
        hct = jnp.transpose(hc)
        hc_ref[...] = jnp.concatenate(
            [hct, jnp.zeros((TOPK, CPAD - NUM_CLASSES), jnp.int32)], axis=1)
        p = p_ref[...]
        n = jnp.sqrt(jnp.sum(p * p, axis=1, keepdims=True))
        pn_ref[0:NUM_CLASSES, :] = p / jnp.maximum(n, 1e-12)
        pn_ref[NUM_CLASSES:CPAD, :] = jnp.zeros(
            (CPAD - NUM_CLASSES, FEATURE_DIM), jnp.float32)

    @pl.when(i > 0)
    def _sim():
        f = f_ref[...]
        n = jnp.sqrt(jnp.sum(f * f, axis=1, keepdims=True))
        fn = f * (1.0 / jnp.maximum(n, 1e-12))
        s = lax.dot_general(fn, pn_ref[...], (((1,), (1,)), ((), ())),
                            preferred_element_type=jnp.float32)
        for g in range(CPAD // 128):
            e_ref[:, g, :] = s[:, g * 128:(g + 1) * 128]


def _sc_gather_body(e_hbm, lab_hbm, hc_hbm, out_hbm, lab_v, hc_v, o_v,
                    idx_refs, val_refs, sem):
    wid = lax.axis_index("s") * 2 + lax.axis_index("c")
    base = wid * RPW
    pltpu.sync_copy(hc_hbm, hc_v)
    pltpu.sync_copy(lab_hbm.at[pl.ds(base, RPW)], lab_v)
    iota = lax.broadcasted_iota(jnp.int32, (16,), 0)
    descs = []
    for j in range(1 + TOPK):
        for g in range(RPW // 16):
            grow = g * 16
            labv = lab_v[pl.ds(grow, 16)]
            rowbase = (base + grow + iota) * CPAD
            if j == 0:
                idx_refs[0][pl.ds(grow, 16)] = rowbase + labv
            else:
                colj = plsc.load_gather(hc_v, [(j - 1) * CPAD + labv])
                idx_refs[j][pl.ds(grow, 16)] = rowbase + colj
        descs.append(pltpu.async_copy(e_hbm.at[idx_refs[j]], val_refs[j], sem))
    for d in descs:
        d.wait()
    for g in range(RPW // 16):
        grow = g * 16
        pos = jnp.exp(val_refs[0][pl.ds(grow, 16)] * (1.0 / TEMP))
        acc = jnp.exp(val_refs[1][pl.ds(grow, 16)] * (1.0 / TEMP))
        for j in range(2, 1 + TOPK):
            acc = acc + jnp.exp(val_refs[j][pl.ds(grow, 16)] * (1.0 / TEMP))
        o_v[pl.ds(grow, 16)] = pos / (pos + acc * (1.0 / TOPK))
    pltpu.sync_copy(o_v, out_hbm.at[wid])


def _loss_body(g_ref, out_ref):
    loss = -jnp.log(g_ref[...])
    out_ref[...] = (jnp.sum(loss) * (1.0 / BATCH)).reshape(1, 1)


def kernel(features, labels, prototypes, confusion_matrix):
    nblk = BATCH // ROW_BLK
    hct, e3 = pl.pallas_call(
        _sim_body,
        grid=(1 + nblk,),
        in_specs=[
            pl.BlockSpec((NUM_CLASSES, NUM_CLASSES), lambda i: (0, 0)),
            pl.BlockSpec((NUM_CLASSES, FEATURE_DIM), lambda i: (0, 0)),
            pl.BlockSpec((ROW_BLK, FEATURE_DIM),
                         lambda i: (jnp.maximum(i - 1, 0), 0)),
        ],
        out_specs=[
            pl.BlockSpec((TOPK, CPAD), lambda i: (0, 0)),
            pl.BlockSpec((ROW_BLK, CPAD // 128, 128),
                         lambda i: (jnp.maximum(i - 1, 0), 0, 0)),
        ],
        out_shape=[jax.ShapeDtypeStruct((TOPK, CPAD), jnp.int32),
                   jax.ShapeDtypeStruct((BATCH, CPAD // 128, 128),
                                        jnp.float32)],
        scratch_shapes=[pltpu.VMEM((CPAD, FEATURE_DIM), jnp.float32)],
    )(confusion_matrix, prototypes, features)
    sc_gather = functools.partial(
        pl.kernel,
        mesh=plsc.VectorSubcoreMesh(core_axis_name="c", subcore_axis_name="s"),
        compiler_params=pltpu.CompilerParams(needs_layout_passes=False),
        out_type=jax.ShapeDtypeStruct((NWORKERS, RPW), jnp.float32),
        scratch_types=[
            pltpu.VMEM((RPW,), jnp.int32),
            pltpu.VMEM((TOPK * CPAD,), jnp.int32),
            pltpu.VMEM((RPW,), jnp.float32),
            [pltpu.VMEM((RPW,), jnp.int32) for _ in range(1 + TOPK)],
            [pltpu.VMEM((RPW,), jnp.float32) for _ in range(1 + TOPK)],
            pltpu.SemaphoreType.DMA,
        ],
    )(_sc_gather_body)
    ratio = sc_gather(e3.reshape(BATCH * CPAD), labels.astype(jnp.int32),
                      hct.reshape(TOPK * CPAD))
    loss = pl.pallas_call(
        _loss_body,
        out_shape=jax.ShapeDtypeStruct((1, 1), jnp.float32),
    )(ratio)
    return loss[0, 0]
